# baseline (speedup 1.0000x reference)
import functools

import jax
import jax.numpy as jnp
import numpy as np
from jax.experimental import pallas as pl
from jax.experimental.pallas import tpu as pltpu

_S = 5
_TINY = np.float32(np.finfo(np.float32).tiny)
_KEY_HI = 0
_KEY_LO = 42


def _rotl(v, d):
    return (v << jnp.uint32(d)) | (v >> jnp.uint32(32 - d))


def _threefry_bits(i_u32):
    ks0 = jnp.uint32(_KEY_HI)
    ks1 = jnp.uint32(_KEY_LO)
    ks2 = jnp.uint32(_KEY_HI ^ _KEY_LO ^ 0x1BD11BDA)
    rot1 = (13, 15, 26, 6)
    rot2 = (17, 29, 16, 24)

    x0 = jnp.full_like(i_u32, ks0)
    x1 = i_u32 + ks1

    def rounds(x0, x1, rots):
        for d in rots:
            x0 = x0 + x1
            x1 = _rotl(x1, d)
            x1 = x1 ^ x0
        return x0, x1

    x0, x1 = rounds(x0, x1, rot1)
    x0 = x0 + ks1
    x1 = x1 + (ks2 + jnp.uint32(1))
    x0, x1 = rounds(x0, x1, rot2)
    x0 = x0 + ks2
    x1 = x1 + (ks0 + jnp.uint32(2))
    x0, x1 = rounds(x0, x1, rot1)
    x0 = x0 + ks0
    x1 = x1 + (ks1 + jnp.uint32(3))
    x0, x1 = rounds(x0, x1, rot2)
    x0 = x0 + ks1
    x1 = x1 + (ks2 + jnp.uint32(4))
    x0, x1 = rounds(x0, x1, rot1)
    x0 = x0 + ks2
    x1 = x1 + (ks0 + jnp.uint32(5))
    return x0 ^ x1


def _gumbel_from_bits(bits):
    fb = (bits >> jnp.uint32(9)) | jnp.uint32(0x3F800000)
    floats = jax.lax.bitcast_convert_type(fb, jnp.float32) - jnp.float32(1.0)
    span = np.float32(np.float32(1.0) - _TINY)
    u = jnp.maximum(jnp.float32(_TINY), floats * span + jnp.float32(_TINY))
    return -jnp.log(-jnp.log(u))


def _body(x_ref, out_ref, best_ref, *, vocab, n_sub, chunk):
    r = pl.program_id(0)
    c = pl.program_id(1)

    xblk = x_ref[0]
    logits = jnp.log(xblk + jnp.float32(1e-30))

    sub = jax.lax.broadcasted_iota(jnp.int32, (n_sub, chunk), 0)
    col = jax.lax.broadcasted_iota(jnp.int32, (n_sub, chunk), 1)
    per_sub = vocab // n_sub
    cglob = c * chunk + col
    valid = cglob < per_sub
    jglob = sub * per_sub + cglob

    for s in range(_S):
        base = (r * _S + s) * vocab
        i = (base + jglob).astype(jnp.uint32)
        g = _gumbel_from_bits(_threefry_bits(i))
        val = jnp.where(valid, logits + g, jnp.float32(-3.0e38))

        m = jnp.max(val)
        idx = jnp.min(jnp.where(val == m, jglob, jnp.int32(0x7FFFFFFF)))

        @pl.when(c == 0)
        def _():
            best_ref[s] = jnp.float32(-3.0e38)

        improve = m > best_ref[s]
        best_ref[s] = jnp.where(improve, m, best_ref[s])
        out_ref[0, 0, s] = jnp.where(improve, idx, out_ref[0, 0, s])


@jax.jit
def kernel(x):
    b, vocab = x.shape
    n_sub = 8
    per_sub = vocab // n_sub
    assert per_sub * n_sub == vocab
    chunk = 1024
    n_chunks = -(-per_sub // chunk)

    x3 = x.reshape(b, n_sub, per_sub)
    out = pl.pallas_call(
        functools.partial(_body, vocab=vocab, n_sub=n_sub, chunk=chunk),
        grid=(b, n_chunks),
        in_specs=[
            pl.BlockSpec((1, n_sub, chunk), lambda r, c: (r, 0, c)),
        ],
        out_specs=pl.BlockSpec((1, 1, _S), lambda r, c: (r, 0, 0),
                               memory_space=pltpu.SMEM),
        out_shape=jax.ShapeDtypeStruct((b, 1, _S), jnp.int32),
        scratch_shapes=[pltpu.SMEM((_S,), jnp.float32)],
    )(x3)
    return out.reshape(b, _S).astype(jnp.int64)


# speedup vs baseline: 1.5642x; 1.5642x over previous
import functools

import jax
import jax.numpy as jnp
import numpy as np
from jax.experimental import pallas as pl
from jax.experimental.pallas import tpu as pltpu

_S = 5
_TINY = np.float32(np.finfo(np.float32).tiny)
_KEY_HI = 0
_KEY_LO = 42


def _rotl(v, d):
    return (v << jnp.uint32(d)) | (v >> jnp.uint32(32 - d))


def _threefry_bits(i_u32):
    ks0 = jnp.uint32(_KEY_HI)
    ks1 = jnp.uint32(_KEY_LO)
    ks2 = jnp.uint32(_KEY_HI ^ _KEY_LO ^ 0x1BD11BDA)
    rot1 = (13, 15, 26, 6)
    rot2 = (17, 29, 16, 24)

    x0 = jnp.full_like(i_u32, ks0)
    x1 = i_u32 + ks1

    def rounds(x0, x1, rots):
        for d in rots:
            x0 = x0 + x1
            x1 = _rotl(x1, d)
            x1 = x1 ^ x0
        return x0, x1

    x0, x1 = rounds(x0, x1, rot1)
    x0 = x0 + ks1
    x1 = x1 + (ks2 + jnp.uint32(1))
    x0, x1 = rounds(x0, x1, rot2)
    x0 = x0 + ks2
    x1 = x1 + (ks0 + jnp.uint32(2))
    x0, x1 = rounds(x0, x1, rot1)
    x0 = x0 + ks0
    x1 = x1 + (ks1 + jnp.uint32(3))
    x0, x1 = rounds(x0, x1, rot2)
    x0 = x0 + ks1
    x1 = x1 + (ks2 + jnp.uint32(4))
    x0, x1 = rounds(x0, x1, rot1)
    x0 = x0 + ks2
    x1 = x1 + (ks0 + jnp.uint32(5))
    return x0 ^ x1


def _gumbel_from_bits(bits):
    fb = (bits >> jnp.uint32(9)) | jnp.uint32(0x3F800000)
    floats = jax.lax.bitcast_convert_type(fb, jnp.float32) - jnp.float32(1.0)
    span = np.float32(np.float32(1.0) - _TINY)
    u = jnp.maximum(jnp.float32(_TINY), floats * span + jnp.float32(_TINY))
    return -jnp.log(-jnp.log(u))


def _body(x_ref, out_ref, vmax_ref, vidx_ref, *, vocab, n_sub, chunk, n_chunks):
    r = pl.program_id(0)
    c = pl.program_id(1)

    xblk = x_ref[0]
    logits = jnp.log(xblk + jnp.float32(1e-30))

    sub = jax.lax.broadcasted_iota(jnp.int32, (n_sub, chunk), 0)
    col = jax.lax.broadcasted_iota(jnp.int32, (n_sub, chunk), 1)
    per_sub = vocab // n_sub
    cglob = c * chunk + col
    valid = cglob < per_sub
    jglob = sub * per_sub + cglob

    for s in range(_S):
        base = (r * _S + s) * vocab
        i = (base + jglob).astype(jnp.uint32)
        g = _gumbel_from_bits(_threefry_bits(i))
        val = jnp.where(valid, logits + g, jnp.float32(-3.0e38))

        @pl.when(c == 0)
        def _():
            vmax_ref[s] = val
            vidx_ref[s] = jglob

        @pl.when(c > 0)
        def _():
            better = val > vmax_ref[s]
            vmax_ref[s] = jnp.where(better, val, vmax_ref[s])
            vidx_ref[s] = jnp.where(better, jglob, vidx_ref[s])

        @pl.when(c == n_chunks - 1)
        def _():
            vm = vmax_ref[s]
            m = jnp.max(vm)
            idx = jnp.min(jnp.where(vm == m, vidx_ref[s],
                                    jnp.int32(0x7FFFFFFF)))
            out_ref[0, 0, s] = idx


@jax.jit
def kernel(x):
    b, vocab = x.shape
    n_sub = 8
    per_sub = vocab // n_sub
    assert per_sub * n_sub == vocab
    chunk = 1024
    n_chunks = -(-per_sub // chunk)

    x3 = x.reshape(b, n_sub, per_sub)
    out = pl.pallas_call(
        functools.partial(_body, vocab=vocab, n_sub=n_sub, chunk=chunk,
                          n_chunks=n_chunks),
        grid=(b, n_chunks),
        in_specs=[
            pl.BlockSpec((1, n_sub, chunk), lambda r, c: (r, 0, c)),
        ],
        out_specs=pl.BlockSpec((1, 1, _S), lambda r, c: (r, 0, 0),
                               memory_space=pltpu.SMEM),
        out_shape=jax.ShapeDtypeStruct((b, 1, _S), jnp.int32),
        scratch_shapes=[
            pltpu.VMEM((_S, n_sub, chunk), jnp.float32),
            pltpu.VMEM((_S, n_sub, chunk), jnp.int32),
        ],
    )(x3)
    return out.reshape(b, _S).astype(jnp.int64)


# speedup vs baseline: 1.7385x; 1.1115x over previous
import functools

import jax
import jax.numpy as jnp
import numpy as np
from jax.experimental import pallas as pl
from jax.experimental.pallas import tpu as pltpu

_S = 5
_TINY = np.float32(np.finfo(np.float32).tiny)
_KEY_HI = 0
_KEY_LO = 42


def _rotl(v, d):
    return (v << jnp.uint32(d)) | (v >> jnp.uint32(32 - d))


def _threefry_bits(i_u32):
    ks0 = jnp.uint32(_KEY_HI)
    ks1 = jnp.uint32(_KEY_LO)
    ks2 = jnp.uint32(_KEY_HI ^ _KEY_LO ^ 0x1BD11BDA)
    rot1 = (13, 15, 26, 6)
    rot2 = (17, 29, 16, 24)

    def rounds(x0, x1, rots):
        for d in rots:
            x0 = x0 + x1
            x1 = _rotl(x1, d)
            x1 = x1 ^ x0
        return x0, x1

    x1 = i_u32 + ks1
    x0 = x1
    x1 = _rotl(x1, rot1[0]) ^ x0
    x0, x1 = rounds(x0, x1, rot1[1:])
    x0 = x0 + ks1
    x1 = x1 + (ks2 + jnp.uint32(1))
    x0, x1 = rounds(x0, x1, rot2)
    x0 = x0 + ks2
    x1 = x1 + jnp.uint32(2)
    x0, x1 = rounds(x0, x1, rot1)
    x1 = x1 + (ks1 + jnp.uint32(3))
    x0, x1 = rounds(x0, x1, rot2)
    x0 = x0 + ks1
    x1 = x1 + (ks2 + jnp.uint32(4))
    x0, x1 = rounds(x0, x1, rot1)
    x0 = x0 + ks2
    x1 = x1 + jnp.uint32(5)
    return x0 ^ x1


def _gumbel_from_bits(bits):
    fb = (bits >> jnp.uint32(9)) | jnp.uint32(0x3F800000)
    floats = jax.lax.bitcast_convert_type(fb, jnp.float32) - jnp.float32(1.0)
    span = np.float32(np.float32(1.0) - _TINY)
    u = jnp.maximum(jnp.float32(_TINY), floats * span + jnp.float32(_TINY))
    return -jnp.log(-jnp.log(u))


def _body(x_ref, out_ref, vmax_ref, vidx_ref, *, vocab, n_sub, chunk, n_chunks):
    r = pl.program_id(0)
    c = pl.program_id(1)

    xblk = x_ref[0]
    logits = jnp.log(xblk + jnp.float32(1e-30))

    sub = jax.lax.broadcasted_iota(jnp.int32, (n_sub, chunk), 0)
    col = jax.lax.broadcasted_iota(jnp.int32, (n_sub, chunk), 1)
    per_sub = vocab // n_sub
    cglob = c * chunk + col
    valid = cglob < per_sub
    jglob = sub * per_sub + cglob

    for s in range(_S):
        base = (r * _S + s) * vocab
        i = (base + jglob).astype(jnp.uint32)
        g = _gumbel_from_bits(_threefry_bits(i))
        val = jnp.where(valid, logits + g, jnp.float32(-3.0e38))

        @pl.when(c == 0)
        def _():
            vmax_ref[s] = val
            vidx_ref[s] = jglob

        @pl.when(c > 0)
        def _():
            better = val > vmax_ref[s]
            vmax_ref[s] = jnp.where(better, val, vmax_ref[s])
            vidx_ref[s] = jnp.where(better, jglob, vidx_ref[s])

        @pl.when(c == n_chunks - 1)
        def _():
            vm = vmax_ref[s]
            m = jnp.max(vm)
            idx = jnp.min(jnp.where(vm == m, vidx_ref[s],
                                    jnp.int32(0x7FFFFFFF)))
            out_ref[0, 0, s] = idx


@jax.jit
def kernel(x):
    b, vocab = x.shape
    n_sub = 8
    per_sub = vocab // n_sub
    assert per_sub * n_sub == vocab
    chunk = 2048
    n_chunks = -(-per_sub // chunk)

    x3 = x.reshape(b, n_sub, per_sub)
    out = pl.pallas_call(
        functools.partial(_body, vocab=vocab, n_sub=n_sub, chunk=chunk,
                          n_chunks=n_chunks),
        grid=(b, n_chunks),
        in_specs=[
            pl.BlockSpec((1, n_sub, chunk), lambda r, c: (r, 0, c)),
        ],
        out_specs=pl.BlockSpec((1, 1, _S), lambda r, c: (r, 0, 0),
                               memory_space=pltpu.SMEM),
        out_shape=jax.ShapeDtypeStruct((b, 1, _S), jnp.int32),
        scratch_shapes=[
            pltpu.VMEM((_S, n_sub, chunk), jnp.float32),
            pltpu.VMEM((_S, n_sub, chunk), jnp.int32),
        ],
    )(x3)
    return out.reshape(b, _S).astype(jnp.int64)
